# Initial kernel scaffold; baseline (speedup 1.0000x reference)
import functools

import jax
import jax.numpy as jnp
from jax import lax
from jax.experimental import pallas as pl
from jax.experimental.pallas import tpu as pltpu
from jax.experimental.pallas import tpu_sc as plsc

_N = 10000
_E = 320000
_IN = 128
_HID = 256
_LAT = 64
_G = 64
_NC, _NS = 2, 16
_NW = _NC * _NS
_EPW = _E // _NW
_CH = 128
_NCHUNK = _EPW // _CH
_TAIL = _EPW - _NCHUNK * _CH
_RPT = _N // _NS
_NP = 10240
_DRPT = _NP // _NS
_BLK = 500
_NB = _N // _BLK
_EPS = 1e-5


def _sc_mesh():
    return plsc.VectorSubcoreMesh(
        core_axis_name="c", subcore_axis_name="s",
        num_cores=_NC, num_subcores=_NS)


@functools.partial(
    pl.kernel,
    out_type=jax.ShapeDtypeStruct((_NC * _NP, 1), jnp.float32),
    mesh=_sc_mesh(),
    scratch_types=[
        pltpu.VMEM_SHARED((_NP, 1), jnp.float32),
        pltpu.VMEM((_CH,), jnp.int32),
        pltpu.VMEM((_CH, 1), jnp.float32),
        pltpu.VMEM((_TAIL,), jnp.int32),
        pltpu.VMEM((_TAIL, 1), jnp.float32),
    ],
)
def _sc_degree(dst_hbm, zeros_hbm, ones_hbm, out_hbm, acc, di, ones_v, dit, ones_t):
    c = lax.axis_index("c")
    s = lax.axis_index("s")
    wid = s * _NC + c
    base = wid * _EPW
    pltpu.sync_copy(zeros_hbm, acc.at[pl.ds(s * _DRPT, _DRPT)])
    pltpu.sync_copy(ones_hbm, ones_v)
    pltpu.sync_copy(ones_hbm.at[pl.ds(0, _TAIL)], ones_t)
    plsc.subcore_barrier()

    def body(i, carry):
        off = base + i * _CH
        pltpu.sync_copy(dst_hbm.at[pl.ds(off, _CH)], di)
        pltpu.sync_copy(ones_v, acc.at[di], add=True)
        return carry

    lax.fori_loop(0, _NCHUNK, body, 0)
    off = base + _NCHUNK * _CH
    pltpu.sync_copy(dst_hbm.at[pl.ds(off, _TAIL)], dit)
    pltpu.sync_copy(ones_t, acc.at[dit], add=True)
    plsc.subcore_barrier()
    pltpu.sync_copy(acc.at[pl.ds(s * _DRPT, _DRPT)],
                    out_hbm.at[pl.ds(c * _NP + s * _DRPT, _DRPT)])


def _make_sc_prop(D):
    @functools.partial(
        pl.kernel,
        out_type=jax.ShapeDtypeStruct((_NC * _N, D), jnp.float32),
        mesh=_sc_mesh(),
        scratch_types=[
            pltpu.VMEM_SHARED((_N, D), jnp.float32),
            pltpu.VMEM((_CH,), jnp.int32),
            pltpu.VMEM((_CH,), jnp.int32),
            pltpu.VMEM((_CH, D), jnp.float32),
            pltpu.VMEM((_TAIL,), jnp.int32),
            pltpu.VMEM((_TAIL,), jnp.int32),
            pltpu.VMEM((_TAIL, D), jnp.float32),
        ],
    )
    def prop(src_hbm, dst_hbm, tab_hbm, zeros_hbm, out_hbm,
             acc, si, di, rows, sit, dit, rows_t):
        c = lax.axis_index("c")
        s = lax.axis_index("s")
        wid = s * _NC + c
        base = wid * _EPW
        pltpu.sync_copy(zeros_hbm, acc.at[pl.ds(s * _RPT, _RPT)])
        plsc.subcore_barrier()

        def body(i, carry):
            off = base + i * _CH
            pltpu.sync_copy(src_hbm.at[pl.ds(off, _CH)], si)
            pltpu.sync_copy(dst_hbm.at[pl.ds(off, _CH)], di)
            pltpu.sync_copy(tab_hbm.at[si], rows)
            pltpu.sync_copy(rows, acc.at[di], add=True)
            return carry

        lax.fori_loop(0, _NCHUNK, body, 0)
        off = base + _NCHUNK * _CH
        pltpu.sync_copy(src_hbm.at[pl.ds(off, _TAIL)], sit)
        pltpu.sync_copy(dst_hbm.at[pl.ds(off, _TAIL)], dit)
        pltpu.sync_copy(tab_hbm.at[sit], rows_t)
        pltpu.sync_copy(rows_t, acc.at[dit], add=True)
        plsc.subcore_barrier()
        pltpu.sync_copy(acc.at[pl.ds(s * _RPT, _RPT)],
                        out_hbm.at[pl.ds(c * _N + s * _RPT, _RPT)])

    return prop


_sc_prop128 = _make_sc_prop(_IN)
_sc_prop64 = _make_sc_prop(_LAT)


def _prescale_body(deg_ref, x_ref, xs_ref, dinv_ref):
    d = deg_ref[0] + deg_ref[1]
    dinv = jnp.where(d > 0, lax.rsqrt(jnp.maximum(d, 1e-12)), 0.0)
    dinv_ref[...] = dinv
    xs_ref[...] = x_ref[...] * dinv


def _tc_prescale(deg2, x):
    return pl.pallas_call(
        _prescale_body,
        grid=(_NB,),
        in_specs=[
            pl.BlockSpec((2, _BLK, 1), lambda i: (0, i, 0)),
            pl.BlockSpec((_BLK, _IN), lambda i: (i, 0)),
        ],
        out_specs=[
            pl.BlockSpec((_BLK, _IN), lambda i: (i, 0)),
            pl.BlockSpec((_BLK, 1), lambda i: (i, 0)),
        ],
        out_shape=[
            jax.ShapeDtypeStruct((_N, _IN), jnp.float32),
            jax.ShapeDtypeStruct((_N, 1), jnp.float32),
        ],
    )(deg2, x)


def _encoder_body(p1_ref, dinv_ref, w1_ref, g1_ref, be1_ref, w2_ref,
                  ts_ref, pst, psum, smat, scsh):
    p = pl.program_id(0)
    i = pl.program_id(1)

    @pl.when(jnp.logical_and(p == 0, i == 0))
    def _():
        psum[...] = jnp.zeros_like(psum)
        smat[...] = jnp.zeros_like(smat)

    @pl.when(p == 0)
    def _():
        pb = (p1_ref[0] + p1_ref[1]) * dinv_ref[...]
        pst[pl.ds(i * _BLK, _BLK), :] = pb
        psum[...] += jnp.sum(pb, axis=0, keepdims=True)
        smat[...] += lax.dot_general(pb, pb, (((0,), (0,)), ((), ())),
                                     preferred_element_type=jnp.float32)

    @pl.when(jnp.logical_and(p == 1, i == 0))
    def _():
        w1 = w1_ref[...]
        pbar = psum[...] * (1.0 / _N)
        mu = lax.dot_general(pbar, w1, (((1,), (0,)), ((), ())),
                             preferred_element_type=jnp.float32)
        cov = smat[...] * (1.0 / _N) - lax.dot_general(
            pbar, pbar, (((0,), (0,)), ((), ())),
            preferred_element_type=jnp.float32)
        cw = lax.dot_general(cov, w1, (((1,), (0,)), ((), ())),
                             preferred_element_type=jnp.float32)
        var = jnp.sum(w1 * cw, axis=0, keepdims=True)
        sc = g1_ref[...] * lax.rsqrt(var + _EPS)
        scsh[0:1, :] = sc
        scsh[1:2, :] = be1_ref[...] - mu * sc

    @pl.when(p == 1)
    def _():
        pb = pst[pl.ds(i * _BLK, _BLK), :]
        h = lax.dot_general(pb, w1_ref[...], (((1,), (0,)), ((), ())),
                            preferred_element_type=jnp.float32)
        z = jnp.maximum(h * scsh[0:1, :] + scsh[1:2, :], 0.0)
        t = lax.dot_general(z, w2_ref[...], (((1,), (0,)), ((), ())),
                            preferred_element_type=jnp.float32)
        ts_ref[...] = t * dinv_ref[...]


def _tc_encoder(p1, dinv, W1, g1, be1, W2):
    return pl.pallas_call(
        _encoder_body,
        grid=(2, _NB),
        in_specs=[
            pl.BlockSpec((2, _BLK, _IN), lambda p, i: (0, i, 0)),
            pl.BlockSpec((_BLK, 1), lambda p, i: (i, 0)),
            pl.BlockSpec((_IN, _HID), lambda p, i: (0, 0)),
            pl.BlockSpec((1, _HID), lambda p, i: (0, 0)),
            pl.BlockSpec((1, _HID), lambda p, i: (0, 0)),
            pl.BlockSpec((_HID, _LAT), lambda p, i: (0, 0)),
        ],
        out_specs=pl.BlockSpec((_BLK, _LAT), lambda p, i: (i, 0)),
        out_shape=jax.ShapeDtypeStruct((_N, _LAT), jnp.float32),
        scratch_shapes=[
            pltpu.VMEM((_N, _IN), jnp.float32),
            pltpu.VMEM((1, _IN), jnp.float32),
            pltpu.VMEM((_IN, _IN), jnp.float32),
            pltpu.VMEM((2, _HID), jnp.float32),
        ],
    )(p1, dinv, W1, g1, be1, W2)


def _decoder_body(p2_ref, dinv_ref, batch_ref, g2_ref, be2_ref,
                  wd1_ref, gd1_ref, bed1_ref, wd2_ref, bd2_ref,
                  xhat_ref, zg_ref,
                  z2st, znst, s2, q2, zsum, szmat, pool, cnt, sc2, scd):
    p = pl.program_id(0)
    i = pl.program_id(1)

    @pl.when(jnp.logical_and(p == 0, i == 0))
    def _():
        s2[...] = jnp.zeros_like(s2)
        q2[...] = jnp.zeros_like(q2)
        zsum[...] = jnp.zeros_like(zsum)
        szmat[...] = jnp.zeros_like(szmat)
        pool[...] = jnp.zeros_like(pool)
        cnt[...] = jnp.zeros_like(cnt)

    @pl.when(p == 0)
    def _():
        z2 = (p2_ref[0] + p2_ref[1]) * dinv_ref[...]
        z2st[pl.ds(i * _BLK, _BLK), :] = z2
        s2[...] += jnp.sum(z2, axis=0, keepdims=True)
        q2[...] += jnp.sum(z2 * z2, axis=0, keepdims=True)

    @pl.when(jnp.logical_and(p == 1, i == 0))
    def _():
        mu2 = s2[...] * (1.0 / _N)
        var2 = q2[...] * (1.0 / _N) - mu2 * mu2
        a = g2_ref[...] * lax.rsqrt(var2 + _EPS)
        sc2[0:1, :] = a
        sc2[1:2, :] = be2_ref[...] - mu2 * a

    @pl.when(p == 1)
    def _():
        zn = jnp.maximum(z2st[pl.ds(i * _BLK, _BLK), :] * sc2[0:1, :]
                         + sc2[1:2, :], 0.0)
        znst[pl.ds(i * _BLK, _BLK), :] = zn
        zsum[...] += jnp.sum(zn, axis=0, keepdims=True)
        szmat[...] += lax.dot_general(zn, zn, (((0,), (0,)), ((), ())),
                                      preferred_element_type=jnp.float32)
        seg = lax.broadcasted_iota(jnp.int32, (1, _G), 1)
        onehot = (batch_ref[...] == seg).astype(jnp.float32)
        pool[...] += lax.dot_general(onehot, zn, (((0,), (0,)), ((), ())),
                                     preferred_element_type=jnp.float32)
        ones = jnp.ones((_BLK, 1), jnp.float32)
        cnt[...] += lax.dot_general(onehot, ones, (((0,), (0,)), ((), ())),
                                    preferred_element_type=jnp.float32)

    @pl.when(jnp.logical_and(p == 2, i == 0))
    def _():
        wd1 = wd1_ref[...]
        zbar = zsum[...] * (1.0 / _N)
        mud = lax.dot_general(zbar, wd1, (((1,), (0,)), ((), ())),
                              preferred_element_type=jnp.float32)
        covz = szmat[...] * (1.0 / _N) - lax.dot_general(
            zbar, zbar, (((0,), (0,)), ((), ())),
            preferred_element_type=jnp.float32)
        cw = lax.dot_general(covz, wd1, (((1,), (0,)), ((), ())),
                             preferred_element_type=jnp.float32)
        vard = jnp.sum(wd1 * cw, axis=0, keepdims=True)
        a = gd1_ref[...] * lax.rsqrt(vard + _EPS)
        scd[0:1, :] = a
        scd[1:2, :] = bed1_ref[...] - mud * a
        zg_ref[...] = pool[...] / jnp.maximum(cnt[...], 1.0)

    @pl.when(p == 2)
    def _():
        zn = znst[pl.ds(i * _BLK, _BLK), :]
        d = lax.dot_general(zn, wd1_ref[...], (((1,), (0,)), ((), ())),
                            preferred_element_type=jnp.float32)
        dn = jnp.maximum(d * scd[0:1, :] + scd[1:2, :], 0.0)
        xhat_ref[...] = lax.dot_general(dn, wd2_ref[...],
                                        (((1,), (0,)), ((), ())),
                                        preferred_element_type=jnp.float32) \
            + bd2_ref[...]


def _tc_decoder(p2, dinv, batch2d, g2, be2, Wd1, gd1, bed1, Wd2, bd2):
    return pl.pallas_call(
        _decoder_body,
        grid=(3, _NB),
        in_specs=[
            pl.BlockSpec((2, _BLK, _LAT), lambda p, i: (0, i, 0)),
            pl.BlockSpec((_BLK, 1), lambda p, i: (i, 0)),
            pl.BlockSpec((_BLK, 1), lambda p, i: (i, 0)),
            pl.BlockSpec((1, _LAT), lambda p, i: (0, 0)),
            pl.BlockSpec((1, _LAT), lambda p, i: (0, 0)),
            pl.BlockSpec((_LAT, _HID), lambda p, i: (0, 0)),
            pl.BlockSpec((1, _HID), lambda p, i: (0, 0)),
            pl.BlockSpec((1, _HID), lambda p, i: (0, 0)),
            pl.BlockSpec((_HID, _IN), lambda p, i: (0, 0)),
            pl.BlockSpec((1, _IN), lambda p, i: (0, 0)),
        ],
        out_specs=[
            pl.BlockSpec((_BLK, _IN), lambda p, i: (i, 0)),
            pl.BlockSpec((_G, _G), lambda p, i: (0, 0)),
        ],
        out_shape=[
            jax.ShapeDtypeStruct((_N, _IN), jnp.float32),
            jax.ShapeDtypeStruct((_G, _G), jnp.float32),
        ],
        scratch_shapes=[
            pltpu.VMEM((_N, _LAT), jnp.float32),
            pltpu.VMEM((_N, _LAT), jnp.float32),
            pltpu.VMEM((1, _LAT), jnp.float32),
            pltpu.VMEM((1, _LAT), jnp.float32),
            pltpu.VMEM((1, _LAT), jnp.float32),
            pltpu.VMEM((_LAT, _LAT), jnp.float32),
            pltpu.VMEM((_G, _G), jnp.float32),
            pltpu.VMEM((_G, 1), jnp.float32),
            pltpu.VMEM((2, _LAT), jnp.float32),
            pltpu.VMEM((2, _HID), jnp.float32),
        ],
    )(p2, dinv, batch2d, g2, be2, Wd1, gd1, bed1, Wd2, bd2)


def kernel(x, edge_index, batch, W1, b1, g1, be1, W2, b2, g2, be2,
           Wd1, bd1, gd1, bed1, Wd2, bd2):
    del b1, b2, bd1
    src = edge_index[0]
    dst = edge_index[1]

    zeros_deg = jnp.zeros((_DRPT, 1), jnp.float32)
    ones_col = jnp.ones((_CH, 1), jnp.float32)
    deg_parts = _sc_degree(dst, zeros_deg, ones_col)
    deg2 = deg_parts.reshape(_NC, _NP, 1)[:, :_N, :]

    xs, dinv = _tc_prescale(deg2, x)

    zeros128 = jnp.zeros((_RPT, _IN), jnp.float32)
    p1 = _sc_prop128(src, dst, xs, zeros128).reshape(_NC, _N, _IN)

    ts = _tc_encoder(p1, dinv, W1,
                     g1.reshape(1, _HID), be1.reshape(1, _HID), W2)

    zeros64 = jnp.zeros((_RPT, _LAT), jnp.float32)
    p2 = _sc_prop64(src, dst, ts, zeros64).reshape(_NC, _N, _LAT)

    x_hat, z_graph = _tc_decoder(
        p2, dinv, batch.reshape(_N, 1),
        g2.reshape(1, _LAT), be2.reshape(1, _LAT),
        Wd1, gd1.reshape(1, _HID), bed1.reshape(1, _HID),
        Wd2, bd2.reshape(1, _IN))
    return (x_hat, z_graph)


# speedup vs baseline: 12.4821x; 12.4821x over previous
import functools

import jax
import jax.numpy as jnp
from jax import lax
from jax.experimental import pallas as pl
from jax.experimental.pallas import tpu as pltpu
from jax.experimental.pallas import tpu_sc as plsc

_N = 10000
_E = 320000
_IN = 128
_HID = 256
_LAT = 64
_G = 64
_NC, _NS = 2, 16
_NW = _NC * _NS
_EPW = _E // _NW
_CH = 128
_NCHUNK = _EPW // _CH
_TAIL = _EPW - _NCHUNK * _CH
_NP = 10240
_DRPT = _NP // _NS
_BLK = 1000
_NB = _N // _BLK
_EPS = 1e-5


def _sc_mesh():
    return plsc.VectorSubcoreMesh(
        core_axis_name="c", subcore_axis_name="s",
        num_cores=_NC, num_subcores=_NS)


def _sc_degree_body(dst_hbm, zeros_hbm, ones_hbm, out_hbm,
                    acc, di, ones_v, dit, ones_t):
    c = lax.axis_index("c")
    s = lax.axis_index("s")
    wid = s * _NC + c
    base = wid * _EPW
    pltpu.sync_copy(zeros_hbm, acc.at[pl.ds(s * _DRPT, _DRPT)])
    pltpu.sync_copy(ones_hbm, ones_v)
    pltpu.sync_copy(ones_hbm.at[pl.ds(0, _TAIL)], ones_t)
    plsc.subcore_barrier()

    def body(i, carry):
        off = base + i * _CH
        pltpu.sync_copy(dst_hbm.at[pl.ds(off, _CH)], di)
        pltpu.sync_copy(ones_v, acc.at[di], add=True)
        return carry

    lax.fori_loop(0, _NCHUNK, body, 0)
    off = base + _NCHUNK * _CH
    pltpu.sync_copy(dst_hbm.at[pl.ds(off, _TAIL)], dit)
    pltpu.sync_copy(ones_t, acc.at[dit], add=True)
    plsc.subcore_barrier()
    pltpu.sync_copy(acc.at[pl.ds(s * _DRPT, _DRPT)],
                    out_hbm.at[pl.ds(c * _NP + s * _DRPT, _DRPT)])


@functools.cache
def _build_sc_degree():
    return pl.kernel(
        _sc_degree_body,
        out_type=jax.ShapeDtypeStruct((_NC * _NP, _IN), jnp.float32),
        mesh=_sc_mesh(),
        scratch_types=[
            pltpu.VMEM_SHARED((_NP, _IN), jnp.float32),
            pltpu.VMEM((_CH,), jnp.int32),
            pltpu.VMEM((_CH, _IN), jnp.float32),
            pltpu.VMEM((_TAIL,), jnp.int32),
            pltpu.VMEM((_TAIL, _IN), jnp.float32),
        ],
    )


def _sc_degree(*args):
    return _build_sc_degree()(*args)


def _make_prop_body(D):
    def prop(src_hbm, dst_hbm, tab_hbm, zeros_hbm, out_hbm,
             acc, si, di, rows, sit, dit, rows_t):
        c = lax.axis_index("c")
        s = lax.axis_index("s")
        wid = s * _NC + c
        base = wid * _EPW
        pltpu.sync_copy(zeros_hbm, acc.at[pl.ds(s * _DRPT, _DRPT)])
        plsc.subcore_barrier()

        def body(i, carry):
            off = base + i * _CH
            pltpu.sync_copy(src_hbm.at[pl.ds(off, _CH)], si)
            pltpu.sync_copy(dst_hbm.at[pl.ds(off, _CH)], di)
            pltpu.sync_copy(tab_hbm.at[si], rows)
            pltpu.sync_copy(rows, acc.at[di], add=True)
            return carry

        lax.fori_loop(0, _NCHUNK, body, 0)
        off = base + _NCHUNK * _CH
        pltpu.sync_copy(src_hbm.at[pl.ds(off, _TAIL)], sit)
        pltpu.sync_copy(dst_hbm.at[pl.ds(off, _TAIL)], dit)
        pltpu.sync_copy(tab_hbm.at[sit], rows_t)
        pltpu.sync_copy(rows_t, acc.at[dit], add=True)
        plsc.subcore_barrier()
        pltpu.sync_copy(acc.at[pl.ds(s * _DRPT, _DRPT)],
                        out_hbm.at[pl.ds(c * _NP + s * _DRPT, _DRPT)])

    return prop


@functools.cache
def _build_sc_prop(D):
    return pl.kernel(
        _make_prop_body(D),
        out_type=jax.ShapeDtypeStruct((_NC * _NP, D), jnp.float32),
        mesh=_sc_mesh(),
        scratch_types=[
            pltpu.VMEM_SHARED((_NP, D), jnp.float32),
            pltpu.VMEM((_CH,), jnp.int32),
            pltpu.VMEM((_CH,), jnp.int32),
            pltpu.VMEM((_CH, D), jnp.float32),
            pltpu.VMEM((_TAIL,), jnp.int32),
            pltpu.VMEM((_TAIL,), jnp.int32),
            pltpu.VMEM((_TAIL, D), jnp.float32),
        ],
    )


def _sc_prop128(*args):
    return _build_sc_prop(_IN)(*args)


def _degsum_body(degp_ref, dinv_ref):
    d = degp_ref[0, :, 0:1] + degp_ref[1, :, 0:1]
    dinv_ref[...] = jnp.where(d > 0, lax.rsqrt(jnp.maximum(d, 1e-12)), 0.0)


def _tc_degsum(degp):
    return pl.pallas_call(
        _degsum_body,
        out_shape=jax.ShapeDtypeStruct((_NP, 1), jnp.float32),
    )(degp)


def _prescale_body(dinv_ref, x_ref, xs_ref):
    xs_ref[...] = x_ref[...] * dinv_ref[...]


def _tc_prescale(dinv_col, x):
    return pl.pallas_call(
        _prescale_body,
        grid=(_NB,),
        in_specs=[
            pl.BlockSpec((_BLK, 1), lambda i: (i, 0)),
            pl.BlockSpec((_BLK, _IN), lambda i: (i, 0)),
        ],
        out_specs=pl.BlockSpec((_BLK, _IN), lambda i: (i, 0)),
        out_shape=jax.ShapeDtypeStruct((_N, _IN), jnp.float32),
    )(dinv_col, x)


def _encoder_body(p1_ref, dinv_ref, w1_ref, g1_ref, be1_ref, w2_ref,
                  ts_ref, pst, psum, smat, scsh):
    p = pl.program_id(0)
    i = pl.program_id(1)

    @pl.when(jnp.logical_and(p == 0, i == 0))
    def _():
        psum[...] = jnp.zeros_like(psum)
        smat[...] = jnp.zeros_like(smat)

    @pl.when(p == 0)
    def _():
        pb = (p1_ref[0] + p1_ref[1]) * dinv_ref[...]
        pst[pl.ds(i * _BLK, _BLK), :] = pb
        psum[...] += jnp.sum(pb, axis=0, keepdims=True)
        smat[...] += lax.dot_general(pb, pb, (((0,), (0,)), ((), ())),
                                     preferred_element_type=jnp.float32)

    @pl.when(jnp.logical_and(p == 1, i == 0))
    def _():
        w1 = w1_ref[...]
        pbar = psum[...] * (1.0 / _N)
        mu = lax.dot_general(pbar, w1, (((1,), (0,)), ((), ())),
                             preferred_element_type=jnp.float32)
        cov = smat[...] * (1.0 / _N) - lax.dot_general(
            pbar, pbar, (((0,), (0,)), ((), ())),
            preferred_element_type=jnp.float32)
        cw = lax.dot_general(cov, w1, (((1,), (0,)), ((), ())),
                             preferred_element_type=jnp.float32)
        var = jnp.sum(w1 * cw, axis=0, keepdims=True)
        sc = g1_ref[...] * lax.rsqrt(var + _EPS)
        scsh[0:1, :] = sc
        scsh[1:2, :] = be1_ref[...] - mu * sc

    @pl.when(p == 1)
    def _():
        pb = pst[pl.ds(i * _BLK, _BLK), :]
        h = lax.dot_general(pb, w1_ref[...], (((1,), (0,)), ((), ())),
                            preferred_element_type=jnp.float32)
        z = jnp.maximum(h * scsh[0:1, :] + scsh[1:2, :], 0.0)
        t = lax.dot_general(z, w2_ref[...], (((1,), (0,)), ((), ())),
                            preferred_element_type=jnp.float32)
        ts_ref[...] = jnp.concatenate(
            [t * dinv_ref[...], jnp.zeros((_BLK, _IN - _LAT), jnp.float32)], 1)


def _tc_encoder(p1, dinv, W1, g1, be1, W2):
    return pl.pallas_call(
        _encoder_body,
        grid=(2, _NB),
        in_specs=[
            pl.BlockSpec((2, _BLK, _IN), lambda p, i: (0, i, 0)),
            pl.BlockSpec((_BLK, 1), lambda p, i: (i, 0)),
            pl.BlockSpec((_IN, _HID), lambda p, i: (0, 0)),
            pl.BlockSpec((1, _HID), lambda p, i: (0, 0)),
            pl.BlockSpec((1, _HID), lambda p, i: (0, 0)),
            pl.BlockSpec((_HID, _LAT), lambda p, i: (0, 0)),
        ],
        out_specs=pl.BlockSpec((_BLK, _IN), lambda p, i: (i, 0)),
        out_shape=jax.ShapeDtypeStruct((_N, _IN), jnp.float32),
        scratch_shapes=[
            pltpu.VMEM((_N, _IN), jnp.float32),
            pltpu.VMEM((1, _IN), jnp.float32),
            pltpu.VMEM((_IN, _IN), jnp.float32),
            pltpu.VMEM((2, _HID), jnp.float32),
        ],
    )(p1, dinv, W1, g1, be1, W2)


def _decoder_body(p2_ref, dinv_ref, batch_ref, g2_ref, be2_ref,
                  wd1_ref, gd1_ref, bed1_ref, wd2_ref, bd2_ref,
                  xhat_ref, zg_ref,
                  z2st, znst, s2, q2, zsum, szmat, pool, cnt, sc2, scd):
    p = pl.program_id(0)
    i = pl.program_id(1)

    @pl.when(jnp.logical_and(p == 0, i == 0))
    def _():
        s2[...] = jnp.zeros_like(s2)
        q2[...] = jnp.zeros_like(q2)
        zsum[...] = jnp.zeros_like(zsum)
        szmat[...] = jnp.zeros_like(szmat)
        pool[...] = jnp.zeros_like(pool)
        cnt[...] = jnp.zeros_like(cnt)

    @pl.when(p == 0)
    def _():
        z2 = (p2_ref[0] + p2_ref[1])[:, :_LAT] * dinv_ref[...]
        z2st[pl.ds(i * _BLK, _BLK), :] = z2
        s2[...] += jnp.sum(z2, axis=0, keepdims=True)
        q2[...] += jnp.sum(z2 * z2, axis=0, keepdims=True)

    @pl.when(jnp.logical_and(p == 1, i == 0))
    def _():
        mu2 = s2[...] * (1.0 / _N)
        var2 = q2[...] * (1.0 / _N) - mu2 * mu2
        a = g2_ref[...] * lax.rsqrt(var2 + _EPS)
        sc2[0:1, :] = a
        sc2[1:2, :] = be2_ref[...] - mu2 * a

    @pl.when(p == 1)
    def _():
        zn = jnp.maximum(z2st[pl.ds(i * _BLK, _BLK), :] * sc2[0:1, :]
                         + sc2[1:2, :], 0.0)
        znst[pl.ds(i * _BLK, _BLK), :] = zn
        zsum[...] += jnp.sum(zn, axis=0, keepdims=True)
        szmat[...] += lax.dot_general(zn, zn, (((0,), (0,)), ((), ())),
                                      preferred_element_type=jnp.float32)
        seg = lax.broadcasted_iota(jnp.int32, (1, _G), 1)
        onehot = (batch_ref[...] == seg).astype(jnp.float32)
        pool[...] += lax.dot_general(onehot, zn, (((0,), (0,)), ((), ())),
                                     preferred_element_type=jnp.float32)
        ones = jnp.ones((_BLK, 1), jnp.float32)
        cnt[...] += lax.dot_general(onehot, ones, (((0,), (0,)), ((), ())),
                                    preferred_element_type=jnp.float32)

    @pl.when(jnp.logical_and(p == 2, i == 0))
    def _():
        wd1 = wd1_ref[...]
        zbar = zsum[...] * (1.0 / _N)
        mud = lax.dot_general(zbar, wd1, (((1,), (0,)), ((), ())),
                              preferred_element_type=jnp.float32)
        covz = szmat[...] * (1.0 / _N) - lax.dot_general(
            zbar, zbar, (((0,), (0,)), ((), ())),
            preferred_element_type=jnp.float32)
        cw = lax.dot_general(covz, wd1, (((1,), (0,)), ((), ())),
                             preferred_element_type=jnp.float32)
        vard = jnp.sum(wd1 * cw, axis=0, keepdims=True)
        a = gd1_ref[...] * lax.rsqrt(vard + _EPS)
        scd[0:1, :] = a
        scd[1:2, :] = bed1_ref[...] - mud * a
        zg_ref[...] = pool[...] / jnp.maximum(cnt[...], 1.0)

    @pl.when(p == 2)
    def _():
        zn = znst[pl.ds(i * _BLK, _BLK), :]
        d = lax.dot_general(zn, wd1_ref[...], (((1,), (0,)), ((), ())),
                            preferred_element_type=jnp.float32)
        dn = jnp.maximum(d * scd[0:1, :] + scd[1:2, :], 0.0)
        xhat_ref[...] = lax.dot_general(dn, wd2_ref[...],
                                        (((1,), (0,)), ((), ())),
                                        preferred_element_type=jnp.float32) \
            + bd2_ref[...]


def _tc_decoder(p2, dinv, batch2d, g2, be2, Wd1, gd1, bed1, Wd2, bd2):
    return pl.pallas_call(
        _decoder_body,
        grid=(3, _NB),
        in_specs=[
            pl.BlockSpec((2, _BLK, _IN), lambda p, i: (0, i, 0)),
            pl.BlockSpec((_BLK, 1), lambda p, i: (i, 0)),
            pl.BlockSpec((_BLK, 1), lambda p, i: (i, 0)),
            pl.BlockSpec((1, _LAT), lambda p, i: (0, 0)),
            pl.BlockSpec((1, _LAT), lambda p, i: (0, 0)),
            pl.BlockSpec((_LAT, _HID), lambda p, i: (0, 0)),
            pl.BlockSpec((1, _HID), lambda p, i: (0, 0)),
            pl.BlockSpec((1, _HID), lambda p, i: (0, 0)),
            pl.BlockSpec((_HID, _IN), lambda p, i: (0, 0)),
            pl.BlockSpec((1, _IN), lambda p, i: (0, 0)),
        ],
        out_specs=[
            pl.BlockSpec((_BLK, _IN), lambda p, i: (i, 0)),
            pl.BlockSpec((_G, _G), lambda p, i: (0, 0)),
        ],
        out_shape=[
            jax.ShapeDtypeStruct((_N, _IN), jnp.float32),
            jax.ShapeDtypeStruct((_G, _G), jnp.float32),
        ],
        scratch_shapes=[
            pltpu.VMEM((_N, _LAT), jnp.float32),
            pltpu.VMEM((_N, _LAT), jnp.float32),
            pltpu.VMEM((1, _LAT), jnp.float32),
            pltpu.VMEM((1, _LAT), jnp.float32),
            pltpu.VMEM((1, _LAT), jnp.float32),
            pltpu.VMEM((_LAT, _LAT), jnp.float32),
            pltpu.VMEM((_G, _G), jnp.float32),
            pltpu.VMEM((_G, 1), jnp.float32),
            pltpu.VMEM((2, _LAT), jnp.float32),
            pltpu.VMEM((2, _HID), jnp.float32),
        ],
    )(p2, dinv, batch2d, g2, be2, Wd1, gd1, bed1, Wd2, bd2)


def kernel(x, edge_index, batch, W1, b1, g1, be1, W2, b2, g2, be2,
           Wd1, bd1, gd1, bed1, Wd2, bd2):
    del b1, b2, bd1
    src = edge_index[0]
    dst = edge_index[1]

    zeros128 = jnp.zeros((_DRPT, _IN), jnp.float32)
    ones128 = jnp.ones((_CH, _IN), jnp.float32)
    degp = _sc_degree(dst, zeros128, ones128).reshape(_NC, _NP, _IN)
    dinv = _tc_degsum(degp)[:_N]
    xs = _tc_prescale(dinv, x)

    p1 = _sc_prop128(src, dst, xs, zeros128).reshape(_NC, _NP, _IN)[:, :_N]

    ts = _tc_encoder(p1, dinv, W1,
                     g1.reshape(1, _HID), be1.reshape(1, _HID), W2)

    p2 = _sc_prop128(src, dst, ts, zeros128).reshape(_NC, _NP, _IN)[:, :_N]

    x_hat, z_graph = _tc_decoder(
        p2, dinv, batch.reshape(_N, 1),
        g2.reshape(1, _LAT), be2.reshape(1, _LAT),
        Wd1, gd1.reshape(1, _HID), bed1.reshape(1, _HID),
        Wd2, bd2.reshape(1, _IN))
    return (x_hat, z_graph)


# speedup vs baseline: 16.8777x; 1.3522x over previous
import functools

import jax
import jax.numpy as jnp
from jax import lax
from jax.experimental import pallas as pl
from jax.experimental.pallas import tpu as pltpu
from jax.experimental.pallas import tpu_sc as plsc

_N = 10000
_E = 320000
_IN = 128
_HID = 256
_LAT = 64
_G = 64
_NC, _NS = 2, 16
_NW = _NC * _NS
_EPW = _E // _NW
_CH = 128
_NCHUNK = _EPW // _CH
_TAIL = _EPW - _NCHUNK * _CH
_NP = 10240
_DRPT = _NP // _NS
_BLK = 1000
_NB = _N // _BLK
_EPS = 1e-5


def _sc_mesh():
    return plsc.VectorSubcoreMesh(
        core_axis_name="c", subcore_axis_name="s",
        num_cores=_NC, num_subcores=_NS)


def _sc_degree_body(dst_hbm, zeros_hbm, ones_hbm, out_hbm,
                    acc, di, ones_v, dit, ones_t):
    c = lax.axis_index("c")
    s = lax.axis_index("s")
    wid = s * _NC + c
    base = wid * _EPW
    pltpu.sync_copy(zeros_hbm, acc.at[pl.ds(s * _DRPT, _DRPT)])
    pltpu.sync_copy(ones_hbm, ones_v)
    pltpu.sync_copy(ones_hbm.at[pl.ds(0, _TAIL)], ones_t)
    plsc.subcore_barrier()

    def body(i, carry):
        off = base + i * _CH
        pltpu.sync_copy(dst_hbm.at[pl.ds(off, _CH)], di)
        pltpu.sync_copy(ones_v, acc.at[di], add=True)
        return carry

    lax.fori_loop(0, _NCHUNK, body, 0)
    off = base + _NCHUNK * _CH
    pltpu.sync_copy(dst_hbm.at[pl.ds(off, _TAIL)], dit)
    pltpu.sync_copy(ones_t, acc.at[dit], add=True)
    plsc.subcore_barrier()
    pltpu.sync_copy(acc.at[pl.ds(s * _DRPT, _DRPT)],
                    out_hbm.at[pl.ds(c * _NP + s * _DRPT, _DRPT)])


@functools.cache
def _build_sc_degree():
    return pl.kernel(
        _sc_degree_body,
        out_type=jax.ShapeDtypeStruct((_NC * _NP, _IN), jnp.float32),
        mesh=_sc_mesh(),
        scratch_types=[
            pltpu.VMEM_SHARED((_NP, _IN), jnp.float32),
            pltpu.VMEM((_CH,), jnp.int32),
            pltpu.VMEM((_CH, _IN), jnp.float32),
            pltpu.VMEM((_TAIL,), jnp.int32),
            pltpu.VMEM((_TAIL, _IN), jnp.float32),
        ],
    )


def _sc_degree(*args):
    return _build_sc_degree()(*args)


def _make_prop_body(D):
    def prop(src_hbm, dst_hbm, tab_hbm, zeros_hbm, out_hbm,
             acc, si2, di2, rows2, sit, dit, rows_t, sem0, sem1):
        c = lax.axis_index("c")
        s = lax.axis_index("s")
        wid = s * _NC + c
        base = wid * _EPW
        pltpu.sync_copy(zeros_hbm, acc.at[pl.ds(s * _DRPT, _DRPT)])
        plsc.subcore_barrier()

        pltpu.sync_copy(src_hbm.at[pl.ds(base, _CH)], si2.at[0])
        pltpu.sync_copy(dst_hbm.at[pl.ds(base, _CH)], di2.at[0])
        pltpu.async_copy(tab_hbm.at[si2.at[0]], rows2.at[0], sem0)

        def half(i, b, semb, nb, semn):
            @pl.when(i + 1 < _NCHUNK)
            def _():
                off = base + (i + 1) * _CH
                pltpu.sync_copy(src_hbm.at[pl.ds(off, _CH)], si2.at[nb])
                pltpu.sync_copy(dst_hbm.at[pl.ds(off, _CH)], di2.at[nb])
                pltpu.async_copy(tab_hbm.at[si2.at[nb]], rows2.at[nb], semn)
            pltpu.make_async_copy(tab_hbm.at[si2.at[b]], rows2.at[b], semb).wait()
            pltpu.sync_copy(rows2.at[b], acc.at[di2.at[b]], add=True)

        def body(j, carry):
            i = j * 2
            half(i, 0, sem0, 1, sem1)
            half(i + 1, 1, sem1, 0, sem0)
            return carry

        lax.fori_loop(0, _NCHUNK // 2, body, 0)
        off = base + _NCHUNK * _CH
        pltpu.sync_copy(src_hbm.at[pl.ds(off, _TAIL)], sit)
        pltpu.sync_copy(dst_hbm.at[pl.ds(off, _TAIL)], dit)
        pltpu.sync_copy(tab_hbm.at[sit], rows_t)
        pltpu.sync_copy(rows_t, acc.at[dit], add=True)
        plsc.subcore_barrier()
        pltpu.sync_copy(acc.at[pl.ds(s * _DRPT, _DRPT)],
                        out_hbm.at[pl.ds(c * _NP + s * _DRPT, _DRPT)])

    return prop


@functools.cache
def _build_sc_prop(D):
    return pl.kernel(
        _make_prop_body(D),
        out_type=jax.ShapeDtypeStruct((_NC * _NP, D), jnp.float32),
        mesh=_sc_mesh(),
        scratch_types=[
            pltpu.VMEM_SHARED((_NP, D), jnp.float32),
            pltpu.VMEM((2, _CH), jnp.int32),
            pltpu.VMEM((2, _CH), jnp.int32),
            pltpu.VMEM((2, _CH, D), jnp.float32),
            pltpu.VMEM((_TAIL,), jnp.int32),
            pltpu.VMEM((_TAIL,), jnp.int32),
            pltpu.VMEM((_TAIL, D), jnp.float32),
            pltpu.SemaphoreType.DMA,
            pltpu.SemaphoreType.DMA,
        ],
    )


def _sc_prop128(*args):
    return _build_sc_prop(_IN)(*args)


def _degsum_body(degp_ref, dinv_ref):
    d = degp_ref[0, :, 0:1] + degp_ref[1, :, 0:1]
    dinv_ref[...] = jnp.where(d > 0, lax.rsqrt(jnp.maximum(d, 1e-12)), 0.0)


def _tc_degsum(degp):
    return pl.pallas_call(
        _degsum_body,
        out_shape=jax.ShapeDtypeStruct((_NP, 1), jnp.float32),
    )(degp)


def _prescale_body(dinv_ref, x_ref, xs_ref):
    xs_ref[...] = x_ref[...] * dinv_ref[...]


def _tc_prescale(dinv_col, x):
    return pl.pallas_call(
        _prescale_body,
        grid=(_NB,),
        in_specs=[
            pl.BlockSpec((_BLK, 1), lambda i: (i, 0)),
            pl.BlockSpec((_BLK, _IN), lambda i: (i, 0)),
        ],
        out_specs=pl.BlockSpec((_BLK, _IN), lambda i: (i, 0)),
        out_shape=jax.ShapeDtypeStruct((_N, _IN), jnp.float32),
    )(dinv_col, x)


def _encoder_body(p1_ref, dinv_ref, w1_ref, g1_ref, be1_ref, w2_ref,
                  ts_ref, pst, psum, smat, scsh):
    p = pl.program_id(0)
    i = pl.program_id(1)

    @pl.when(jnp.logical_and(p == 0, i == 0))
    def _():
        psum[...] = jnp.zeros_like(psum)
        smat[...] = jnp.zeros_like(smat)

    @pl.when(p == 0)
    def _():
        pb = (p1_ref[0] + p1_ref[1]) * dinv_ref[...]
        pst[pl.ds(i * _BLK, _BLK), :] = pb
        psum[...] += jnp.sum(pb, axis=0, keepdims=True)
        smat[...] += lax.dot_general(pb, pb, (((0,), (0,)), ((), ())),
                                     preferred_element_type=jnp.float32)

    @pl.when(jnp.logical_and(p == 1, i == 0))
    def _():
        w1 = w1_ref[...]
        pbar = psum[...] * (1.0 / _N)
        mu = lax.dot_general(pbar, w1, (((1,), (0,)), ((), ())),
                             preferred_element_type=jnp.float32)
        cov = smat[...] * (1.0 / _N) - lax.dot_general(
            pbar, pbar, (((0,), (0,)), ((), ())),
            preferred_element_type=jnp.float32)
        cw = lax.dot_general(cov, w1, (((1,), (0,)), ((), ())),
                             preferred_element_type=jnp.float32)
        var = jnp.sum(w1 * cw, axis=0, keepdims=True)
        sc = g1_ref[...] * lax.rsqrt(var + _EPS)
        scsh[0:1, :] = sc
        scsh[1:2, :] = be1_ref[...] - mu * sc

    @pl.when(p == 1)
    def _():
        pb = pst[pl.ds(i * _BLK, _BLK), :]
        h = lax.dot_general(pb, w1_ref[...], (((1,), (0,)), ((), ())),
                            preferred_element_type=jnp.float32)
        z = jnp.maximum(h * scsh[0:1, :] + scsh[1:2, :], 0.0)
        t = lax.dot_general(z, w2_ref[...], (((1,), (0,)), ((), ())),
                            preferred_element_type=jnp.float32)
        ts_ref[...] = jnp.concatenate(
            [t * dinv_ref[...], jnp.zeros((_BLK, _IN - _LAT), jnp.float32)], 1)


def _tc_encoder(p1, dinv, W1, g1, be1, W2):
    return pl.pallas_call(
        _encoder_body,
        grid=(2, _NB),
        in_specs=[
            pl.BlockSpec((2, _BLK, _IN), lambda p, i: (0, i, 0)),
            pl.BlockSpec((_BLK, 1), lambda p, i: (i, 0)),
            pl.BlockSpec((_IN, _HID), lambda p, i: (0, 0)),
            pl.BlockSpec((1, _HID), lambda p, i: (0, 0)),
            pl.BlockSpec((1, _HID), lambda p, i: (0, 0)),
            pl.BlockSpec((_HID, _LAT), lambda p, i: (0, 0)),
        ],
        out_specs=pl.BlockSpec((_BLK, _IN), lambda p, i: (i, 0)),
        out_shape=jax.ShapeDtypeStruct((_N, _IN), jnp.float32),
        scratch_shapes=[
            pltpu.VMEM((_N, _IN), jnp.float32),
            pltpu.VMEM((1, _IN), jnp.float32),
            pltpu.VMEM((_IN, _IN), jnp.float32),
            pltpu.VMEM((2, _HID), jnp.float32),
        ],
    )(p1, dinv, W1, g1, be1, W2)


def _decoder_body(p2_ref, dinv_ref, batch_ref, g2_ref, be2_ref,
                  wd1_ref, gd1_ref, bed1_ref, wd2_ref, bd2_ref,
                  xhat_ref, zg_ref,
                  z2st, znst, s2, q2, zsum, szmat, pool, cnt, sc2, scd):
    p = pl.program_id(0)
    i = pl.program_id(1)

    @pl.when(jnp.logical_and(p == 0, i == 0))
    def _():
        s2[...] = jnp.zeros_like(s2)
        q2[...] = jnp.zeros_like(q2)
        zsum[...] = jnp.zeros_like(zsum)
        szmat[...] = jnp.zeros_like(szmat)
        pool[...] = jnp.zeros_like(pool)
        cnt[...] = jnp.zeros_like(cnt)

    @pl.when(p == 0)
    def _():
        z2 = (p2_ref[0] + p2_ref[1])[:, :_LAT] * dinv_ref[...]
        z2st[pl.ds(i * _BLK, _BLK), :] = z2
        s2[...] += jnp.sum(z2, axis=0, keepdims=True)
        q2[...] += jnp.sum(z2 * z2, axis=0, keepdims=True)

    @pl.when(jnp.logical_and(p == 1, i == 0))
    def _():
        mu2 = s2[...] * (1.0 / _N)
        var2 = q2[...] * (1.0 / _N) - mu2 * mu2
        a = g2_ref[...] * lax.rsqrt(var2 + _EPS)
        sc2[0:1, :] = a
        sc2[1:2, :] = be2_ref[...] - mu2 * a

    @pl.when(p == 1)
    def _():
        zn = jnp.maximum(z2st[pl.ds(i * _BLK, _BLK), :] * sc2[0:1, :]
                         + sc2[1:2, :], 0.0)
        znst[pl.ds(i * _BLK, _BLK), :] = zn
        zsum[...] += jnp.sum(zn, axis=0, keepdims=True)
        szmat[...] += lax.dot_general(zn, zn, (((0,), (0,)), ((), ())),
                                      preferred_element_type=jnp.float32)
        seg = lax.broadcasted_iota(jnp.int32, (1, _G), 1)
        onehot = (batch_ref[...] == seg).astype(jnp.float32)
        pool[...] += lax.dot_general(onehot, zn, (((0,), (0,)), ((), ())),
                                     preferred_element_type=jnp.float32)
        ones = jnp.ones((_BLK, 1), jnp.float32)
        cnt[...] += lax.dot_general(onehot, ones, (((0,), (0,)), ((), ())),
                                    preferred_element_type=jnp.float32)

    @pl.when(jnp.logical_and(p == 2, i == 0))
    def _():
        wd1 = wd1_ref[...]
        zbar = zsum[...] * (1.0 / _N)
        mud = lax.dot_general(zbar, wd1, (((1,), (0,)), ((), ())),
                              preferred_element_type=jnp.float32)
        covz = szmat[...] * (1.0 / _N) - lax.dot_general(
            zbar, zbar, (((0,), (0,)), ((), ())),
            preferred_element_type=jnp.float32)
        cw = lax.dot_general(covz, wd1, (((1,), (0,)), ((), ())),
                             preferred_element_type=jnp.float32)
        vard = jnp.sum(wd1 * cw, axis=0, keepdims=True)
        a = gd1_ref[...] * lax.rsqrt(vard + _EPS)
        scd[0:1, :] = a
        scd[1:2, :] = bed1_ref[...] - mud * a
        zg_ref[...] = pool[...] / jnp.maximum(cnt[...], 1.0)

    @pl.when(p == 2)
    def _():
        zn = znst[pl.ds(i * _BLK, _BLK), :]
        d = lax.dot_general(zn, wd1_ref[...], (((1,), (0,)), ((), ())),
                            preferred_element_type=jnp.float32)
        dn = jnp.maximum(d * scd[0:1, :] + scd[1:2, :], 0.0)
        xhat_ref[...] = lax.dot_general(dn, wd2_ref[...],
                                        (((1,), (0,)), ((), ())),
                                        preferred_element_type=jnp.float32) \
            + bd2_ref[...]


def _tc_decoder(p2, dinv, batch2d, g2, be2, Wd1, gd1, bed1, Wd2, bd2):
    return pl.pallas_call(
        _decoder_body,
        grid=(3, _NB),
        in_specs=[
            pl.BlockSpec((2, _BLK, _IN), lambda p, i: (0, i, 0)),
            pl.BlockSpec((_BLK, 1), lambda p, i: (i, 0)),
            pl.BlockSpec((_BLK, 1), lambda p, i: (i, 0)),
            pl.BlockSpec((1, _LAT), lambda p, i: (0, 0)),
            pl.BlockSpec((1, _LAT), lambda p, i: (0, 0)),
            pl.BlockSpec((_LAT, _HID), lambda p, i: (0, 0)),
            pl.BlockSpec((1, _HID), lambda p, i: (0, 0)),
            pl.BlockSpec((1, _HID), lambda p, i: (0, 0)),
            pl.BlockSpec((_HID, _IN), lambda p, i: (0, 0)),
            pl.BlockSpec((1, _IN), lambda p, i: (0, 0)),
        ],
        out_specs=[
            pl.BlockSpec((_BLK, _IN), lambda p, i: (i, 0)),
            pl.BlockSpec((_G, _G), lambda p, i: (0, 0)),
        ],
        out_shape=[
            jax.ShapeDtypeStruct((_N, _IN), jnp.float32),
            jax.ShapeDtypeStruct((_G, _G), jnp.float32),
        ],
        scratch_shapes=[
            pltpu.VMEM((_N, _LAT), jnp.float32),
            pltpu.VMEM((_N, _LAT), jnp.float32),
            pltpu.VMEM((1, _LAT), jnp.float32),
            pltpu.VMEM((1, _LAT), jnp.float32),
            pltpu.VMEM((1, _LAT), jnp.float32),
            pltpu.VMEM((_LAT, _LAT), jnp.float32),
            pltpu.VMEM((_G, _G), jnp.float32),
            pltpu.VMEM((_G, 1), jnp.float32),
            pltpu.VMEM((2, _LAT), jnp.float32),
            pltpu.VMEM((2, _HID), jnp.float32),
        ],
    )(p2, dinv, batch2d, g2, be2, Wd1, gd1, bed1, Wd2, bd2)


def kernel(x, edge_index, batch, W1, b1, g1, be1, W2, b2, g2, be2,
           Wd1, bd1, gd1, bed1, Wd2, bd2):
    del b1, b2, bd1
    src = edge_index[0]
    dst = edge_index[1]

    zeros128 = jnp.zeros((_DRPT, _IN), jnp.float32)
    ones128 = jnp.ones((_CH, _IN), jnp.float32)
    degp = _sc_degree(dst, zeros128, ones128).reshape(_NC, _NP, _IN)
    dinv = _tc_degsum(degp)[:_N]
    xs = _tc_prescale(dinv, x)

    p1 = _sc_prop128(src, dst, xs, zeros128).reshape(_NC, _NP, _IN)[:, :_N]

    ts = _tc_encoder(p1, dinv, W1,
                     g1.reshape(1, _HID), be1.reshape(1, _HID), W2)

    p2 = _sc_prop128(src, dst, ts, zeros128).reshape(_NC, _NP, _IN)[:, :_N]

    x_hat, z_graph = _tc_decoder(
        p2, dinv, batch.reshape(_N, 1),
        g2.reshape(1, _LAT), be2.reshape(1, _LAT),
        Wd1, gd1.reshape(1, _HID), bed1.reshape(1, _HID),
        Wd2, bd2.reshape(1, _IN))
    return (x_hat, z_graph)
